# Initial kernel scaffold; baseline (speedup 1.0000x reference)
import jax
import jax.numpy as jnp
from jax.experimental import pallas as pl


def kernel(e1, rel, edge_index, emb_e, emb_rel, W_msg, W_ih, W_hh, b_ih, b_hh):
    raise NotImplementedError("write your pallas kernel here")


# speedup vs baseline: 3.3277x; 3.3277x over previous
import functools

import jax
import jax.numpy as jnp
from jax import lax
from jax.experimental import pallas as pl
from jax.experimental.pallas import tpu as pltpu
from jax.experimental.pallas import tpu_sc as plsc

N_ENT = 10000
N_REL = 200
D = 128
E = 320000
B = 1024

NC = 2
NS = 16
NW = NC * NS
CH = 128
EPW = -(-E // (NW * CH)) * CH
EPAD = EPW * NW
NCH = EPW // CH
NPAD = 10112
RPT = NPAD // NS
BPW = B // NW

ROWBLK = 2000
EBLK = 1280
NEPAD = 10240

_mesh = plsc.VectorSubcoreMesh(core_axis_name="c", subcore_axis_name="s")


@functools.partial(
    pl.kernel,
    out_type=jax.ShapeDtypeStruct((NC, NPAD, D), jnp.float32),
    mesh=_mesh,
    scratch_types=[
        pltpu.VMEM((CH, D), jnp.float32),
        pltpu.VMEM((CH,), jnp.int32),
        pltpu.VMEM((CH,), jnp.int32),
        pltpu.VMEM_SHARED((NPAD, D), jnp.float32),
        pltpu.SemaphoreType.DMA,
    ],
)
def _sc_segsum(hw_hbm, src_hbm, dst_hbm, zeros_hbm, out_hbm,
               rows_v, src_v, dst_v, acc, sem):
    c = lax.axis_index("c")
    s = lax.axis_index("s")
    wid = s * NC + c

    pltpu.sync_copy(zeros_hbm.at[pl.ds(s * RPT, RPT)], acc.at[pl.ds(s * RPT, RPT)])
    plsc.subcore_barrier()

    def chunk(j, carry):
        pltpu.sync_copy(src_hbm.at[wid, j], src_v)
        pltpu.sync_copy(dst_hbm.at[wid, j], dst_v)
        pltpu.async_copy(hw_hbm.at[src_v], rows_v, sem).wait()
        pltpu.sync_copy(rows_v, acc.at[dst_v], add=True)
        return carry

    lax.fori_loop(0, NCH, chunk, 0)
    plsc.subcore_barrier()
    pltpu.sync_copy(acc.at[pl.ds(s * RPT, RPT)], out_hbm.at[c, pl.ds(s * RPT, RPT)])


@functools.partial(
    pl.kernel,
    out_type=[
        jax.ShapeDtypeStruct((B, D), jnp.float32),
        jax.ShapeDtypeStruct((B, D), jnp.float32),
    ],
    mesh=_mesh,
    scratch_types=[
        pltpu.VMEM((BPW,), jnp.int32),
        pltpu.VMEM((BPW,), jnp.int32),
        pltpu.VMEM((BPW, D), jnp.float32),
        pltpu.VMEM((BPW, D), jnp.float32),
        pltpu.SemaphoreType.DMA,
    ],
)
def _sc_qgather(h_hbm, erel_hbm, e1_hbm, rel_hbm, o1_hbm, o2_hbm,
                i1_v, i2_v, r1_v, r2_v, sem):
    c = lax.axis_index("c")
    s = lax.axis_index("s")
    wid = s * NC + c
    pltpu.sync_copy(e1_hbm.at[wid], i1_v)
    pltpu.sync_copy(rel_hbm.at[wid], i2_v)
    pltpu.async_copy(h_hbm.at[i1_v], r1_v, sem).wait()
    pltpu.async_copy(erel_hbm.at[i2_v], r2_v, sem).wait()
    pltpu.sync_copy(r1_v, o1_hbm.at[pl.ds(wid * BPW, BPW)])
    pltpu.sync_copy(r2_v, o2_hbm.at[pl.ds(wid * BPW, BPW)])


def _mm_body(x_ref, w_ref, o_ref):
    o_ref[...] = jnp.dot(x_ref[...], w_ref[...], preferred_element_type=jnp.float32)


def _tc_matmul(x, w):
    n = x.shape[0]
    return pl.pallas_call(
        _mm_body,
        grid=(n // ROWBLK,),
        in_specs=[
            pl.BlockSpec((ROWBLK, D), lambda i: (i, 0)),
            pl.BlockSpec((D, D), lambda i: (0, 0)),
        ],
        out_specs=pl.BlockSpec((ROWBLK, D), lambda i: (i, 0)),
        out_shape=jax.ShapeDtypeStruct((n, D), jnp.float32),
    )(x, w)


def _gru_body(p_ref, h_ref, wih_ref, whh_ref, bih_ref, bhh_ref, wmsg_ref,
              h_out, hw_out):
    a = p_ref[0] + p_ref[1]
    h = h_ref[...]
    gi = jnp.dot(a, wih_ref[...], preferred_element_type=jnp.float32) + bih_ref[...]
    gh = jnp.dot(h, whh_ref[...], preferred_element_type=jnp.float32) + bhh_ref[...]
    r = jax.nn.sigmoid(gi[:, :D] + gh[:, :D])
    z = jax.nn.sigmoid(gi[:, D:2 * D] + gh[:, D:2 * D])
    n = jnp.tanh(gi[:, 2 * D:] + r * gh[:, 2 * D:])
    hn = (1.0 - z) * n + z * h
    h_out[...] = hn
    hw_out[...] = jnp.dot(hn, wmsg_ref[...], preferred_element_type=jnp.float32)


def _tc_gru(parts, h, w_ih, w_hh, b_ih, b_hh, w_msg):
    return pl.pallas_call(
        _gru_body,
        grid=(N_ENT // ROWBLK,),
        in_specs=[
            pl.BlockSpec((NC, ROWBLK, D), lambda i: (0, i, 0)),
            pl.BlockSpec((ROWBLK, D), lambda i: (i, 0)),
            pl.BlockSpec((D, 3 * D), lambda i: (0, 0)),
            pl.BlockSpec((D, 3 * D), lambda i: (0, 0)),
            pl.BlockSpec((1, 3 * D), lambda i: (0, 0)),
            pl.BlockSpec((1, 3 * D), lambda i: (0, 0)),
            pl.BlockSpec((D, D), lambda i: (0, 0)),
        ],
        out_specs=[
            pl.BlockSpec((ROWBLK, D), lambda i: (i, 0)),
            pl.BlockSpec((ROWBLK, D), lambda i: (i, 0)),
        ],
        out_shape=[
            jax.ShapeDtypeStruct((N_ENT, D), jnp.float32),
            jax.ShapeDtypeStruct((N_ENT, D), jnp.float32),
        ],
    )(parts, h, w_ih, w_hh, b_ih, b_hh, w_msg)


def _score_body(q1_ref, q2_ref, e_ref, o_ref):
    q = q1_ref[...] * q2_ref[...]
    o_ref[...] = jax.nn.sigmoid(
        lax.dot_general(q, e_ref[...], (((1,), (1,)), ((), ())),
                        preferred_element_type=jnp.float32))


def _tc_score(q1, q2, emb_pad):
    return pl.pallas_call(
        _score_body,
        grid=(NEPAD // EBLK,),
        in_specs=[
            pl.BlockSpec((B, D), lambda i: (0, 0)),
            pl.BlockSpec((B, D), lambda i: (0, 0)),
            pl.BlockSpec((EBLK, D), lambda i: (i, 0)),
        ],
        out_specs=pl.BlockSpec((B, EBLK), lambda i: (0, i)),
        out_shape=jax.ShapeDtypeStruct((B, NEPAD), jnp.float32),
    )(q1, q2, emb_pad)


def kernel(e1, rel, edge_index, emb_e, emb_rel, W_msg, W_ih, W_hh, b_ih, b_hh):
    src = edge_index[0].astype(jnp.int32)
    dst = edge_index[1].astype(jnp.int32)
    npad = EPAD - E
    srcp = jnp.concatenate([src, jnp.zeros((npad,), jnp.int32)]).reshape(NW, NCH, CH)
    dstp = jnp.concatenate([dst, jnp.full((npad,), N_ENT, jnp.int32)]).reshape(NW, NCH, CH)
    zeros = jnp.zeros((NPAD, D), jnp.float32)
    bih2 = b_ih.reshape(1, 3 * D)
    bhh2 = b_hh.reshape(1, 3 * D)
    e1i = e1[:, 0].astype(jnp.int32).reshape(NW, BPW)
    reli = rel[:, 0].astype(jnp.int32).reshape(NW, BPW)

    h = emb_e
    hw = _tc_matmul(h, W_msg)
    for _ in range(2):
        parts = _sc_segsum(hw, srcp, dstp, zeros)
        h, hw = _tc_gru(parts[:, :N_ENT, :], h, W_ih, W_hh, bih2, bhh2, W_msg)

    e1r, relr = _sc_qgather(h, emb_rel, e1i, reli)
    emb_pad = jnp.concatenate(
        [emb_e, jnp.zeros((NEPAD - N_ENT, D), jnp.float32)], axis=0)
    logits = _tc_score(e1r, relr, emb_pad)
    return logits[:, :N_ENT]


# speedup vs baseline: 3.3291x; 1.0004x over previous
import functools

import jax
import jax.numpy as jnp
from jax import lax
from jax.experimental import pallas as pl
from jax.experimental.pallas import tpu as pltpu
from jax.experimental.pallas import tpu_sc as plsc

N_ENT = 10000
N_REL = 200
D = 128
E = 320000
B = 1024

NC = 2
NS = 16
NW = NC * NS
CH = 128
EPW = -(-E // (NW * CH)) * CH
EPAD = EPW * NW
NCH = EPW // CH
NPAD = 10112
RPT = NPAD // NS
BPW = B // NW

ROWBLK = 2000
EBLK = 1280
NEPAD = 10240

_mesh = plsc.VectorSubcoreMesh(core_axis_name="c", subcore_axis_name="s")


@functools.partial(
    pl.kernel,
    out_type=jax.ShapeDtypeStruct((NC, NPAD, D), jnp.float32),
    mesh=_mesh,
    scratch_types=[
        pltpu.VMEM((CH,), jnp.int32),
        pltpu.VMEM((CH,), jnp.int32),
        pltpu.VMEM((CH, D), jnp.float32),
        pltpu.VMEM_SHARED((NPAD, D), jnp.float32),
        pltpu.SemaphoreType.DMA,
    ],
)
def _sc_segsum(hw_hbm, src_hbm, dst_hbm, zeros_hbm, out_hbm,
               is0, id0, r0, acc, sg0):
    c = lax.axis_index("c")
    s = lax.axis_index("s")
    wid = s * NC + c

    pltpu.sync_copy(zeros_hbm.at[pl.ds(s * RPT, RPT)], acc.at[pl.ds(s * RPT, RPT)])
    plsc.subcore_barrier()

    def body(j, carry):
        pltpu.sync_copy(src_hbm.at[wid, j], is0)
        pltpu.sync_copy(dst_hbm.at[wid, j], id0)
        pltpu.async_copy(hw_hbm.at[is0], r0, sg0).wait()
        pltpu.sync_copy(r0, acc.at[id0], add=True)
        return carry

    lax.fori_loop(0, NCH, body, 0)
    plsc.subcore_barrier()
    pltpu.sync_copy(acc.at[pl.ds(s * RPT, RPT)], out_hbm.at[c, pl.ds(s * RPT, RPT)])


@functools.partial(
    pl.kernel,
    out_type=[
        jax.ShapeDtypeStruct((B, D), jnp.float32),
        jax.ShapeDtypeStruct((B, D), jnp.float32),
    ],
    mesh=_mesh,
    scratch_types=[
        pltpu.VMEM((BPW,), jnp.int32),
        pltpu.VMEM((BPW,), jnp.int32),
        pltpu.VMEM((BPW, D), jnp.float32),
        pltpu.VMEM((BPW, D), jnp.float32),
        pltpu.SemaphoreType.DMA,
    ],
)
def _sc_qgather(h_hbm, erel_hbm, e1_hbm, rel_hbm, o1_hbm, o2_hbm,
                i1_v, i2_v, r1_v, r2_v, sem):
    c = lax.axis_index("c")
    s = lax.axis_index("s")
    wid = s * NC + c
    pltpu.sync_copy(e1_hbm.at[wid], i1_v)
    pltpu.sync_copy(rel_hbm.at[wid], i2_v)
    pltpu.async_copy(h_hbm.at[i1_v], r1_v, sem).wait()
    pltpu.async_copy(erel_hbm.at[i2_v], r2_v, sem).wait()
    pltpu.sync_copy(r1_v, o1_hbm.at[pl.ds(wid * BPW, BPW)])
    pltpu.sync_copy(r2_v, o2_hbm.at[pl.ds(wid * BPW, BPW)])


def _mm_body(x_ref, w_ref, o_ref):
    o_ref[...] = jnp.dot(x_ref[...], w_ref[...],
                         preferred_element_type=jnp.float32)


def _tc_matmul(x, w):
    n = x.shape[0]
    return pl.pallas_call(
        _mm_body,
        grid=(n // ROWBLK,),
        in_specs=[
            pl.BlockSpec((ROWBLK, D), lambda i: (i, 0)),
            pl.BlockSpec((D, D), lambda i: (0, 0)),
        ],
        out_specs=pl.BlockSpec((ROWBLK, D), lambda i: (i, 0)),
        out_shape=jax.ShapeDtypeStruct((n, D), jnp.float32),
    )(x, w)


def _gru_body(p_ref, h_ref, wih_ref, whh_ref, bih_ref, bhh_ref, wmsg_ref,
              h_out, hw_out):
    a = p_ref[0] + p_ref[1]
    h = h_ref[...]
    gi = jnp.dot(a, wih_ref[...], preferred_element_type=jnp.float32) + bih_ref[...]
    gh = jnp.dot(h, whh_ref[...], preferred_element_type=jnp.float32) + bhh_ref[...]
    r = jax.nn.sigmoid(gi[:, :D] + gh[:, :D])
    z = jax.nn.sigmoid(gi[:, D:2 * D] + gh[:, D:2 * D])
    n = jnp.tanh(gi[:, 2 * D:] + r * gh[:, 2 * D:])
    hn = (1.0 - z) * n + z * h
    h_out[...] = hn
    hw_out[...] = jnp.dot(hn, wmsg_ref[...], preferred_element_type=jnp.float32)


def _tc_gru(parts, h, w_ih, w_hh, b_ih, b_hh, w_msg):
    return pl.pallas_call(
        _gru_body,
        grid=(N_ENT // ROWBLK,),
        in_specs=[
            pl.BlockSpec((NC, ROWBLK, D), lambda i: (0, i, 0)),
            pl.BlockSpec((ROWBLK, D), lambda i: (i, 0)),
            pl.BlockSpec((D, 3 * D), lambda i: (0, 0)),
            pl.BlockSpec((D, 3 * D), lambda i: (0, 0)),
            pl.BlockSpec((1, 3 * D), lambda i: (0, 0)),
            pl.BlockSpec((1, 3 * D), lambda i: (0, 0)),
            pl.BlockSpec((D, D), lambda i: (0, 0)),
        ],
        out_specs=[
            pl.BlockSpec((ROWBLK, D), lambda i: (i, 0)),
            pl.BlockSpec((ROWBLK, D), lambda i: (i, 0)),
        ],
        out_shape=[
            jax.ShapeDtypeStruct((N_ENT, D), jnp.float32),
            jax.ShapeDtypeStruct((N_ENT, D), jnp.float32),
        ],
    )(parts, h, w_ih, w_hh, b_ih, b_hh, w_msg)


def _score_body(q1_ref, q2_ref, e_ref, o_ref):
    q = q1_ref[...] * q2_ref[...]
    o_ref[...] = jax.nn.sigmoid(
        lax.dot_general(q, e_ref[...], (((1,), (1,)), ((), ())),
                        preferred_element_type=jnp.float32))


def _tc_score(q1, q2, emb_pad):
    return pl.pallas_call(
        _score_body,
        grid=(NEPAD // EBLK,),
        in_specs=[
            pl.BlockSpec((B, D), lambda i: (0, 0)),
            pl.BlockSpec((B, D), lambda i: (0, 0)),
            pl.BlockSpec((EBLK, D), lambda i: (i, 0)),
        ],
        out_specs=pl.BlockSpec((B, EBLK), lambda i: (0, i)),
        out_shape=jax.ShapeDtypeStruct((B, NEPAD), jnp.float32),
    )(q1, q2, emb_pad)


def kernel(e1, rel, edge_index, emb_e, emb_rel, W_msg, W_ih, W_hh, b_ih, b_hh):
    src = edge_index[0].astype(jnp.int32)
    dst = edge_index[1].astype(jnp.int32)
    npad = EPAD - E
    srcp = jnp.concatenate([src, jnp.zeros((npad,), jnp.int32)]).reshape(NW, NCH, CH)
    dstp = jnp.concatenate([dst, jnp.full((npad,), N_ENT, jnp.int32)]).reshape(NW, NCH, CH)
    zeros = jnp.zeros((NPAD, D), jnp.float32)
    bih2 = b_ih.reshape(1, 3 * D)
    bhh2 = b_hh.reshape(1, 3 * D)
    e1i = e1[:, 0].astype(jnp.int32).reshape(NW, BPW)
    reli = rel[:, 0].astype(jnp.int32).reshape(NW, BPW)

    h = emb_e
    hw = _tc_matmul(h, W_msg)
    for _ in range(2):
        parts = _sc_segsum(hw, srcp, dstp, zeros)
        h, hw = _tc_gru(parts[:, :N_ENT, :], h, W_ih, W_hh, bih2, bhh2, W_msg)

    e1r, relr = _sc_qgather(h, emb_rel, e1i, reli)
    emb_pad = jnp.concatenate(
        [emb_e, jnp.zeros((NEPAD - N_ENT, D), jnp.float32)], axis=0)
    return _tc_score(e1r, relr, emb_pad)[:, :N_ENT]


# speedup vs baseline: 4.1043x; 1.2329x over previous
import functools

import jax
import jax.numpy as jnp
from jax import lax
from jax.experimental import pallas as pl
from jax.experimental.pallas import tpu as pltpu
from jax.experimental.pallas import tpu_sc as plsc

N_ENT = 10000
N_REL = 200
D = 128
E = 320000
B = 1024

NC = 2
NS = 16
NW = NC * NS
CH = 128
EPW = -(-E // (NW * CH)) * CH
EPAD = EPW * NW
NCH = EPW // CH
NPAD = 10112
RPT = NPAD // NS
BPW = B // NW

ROWBLK = 2000
EBLK = 1280
NEPAD = 10240

_mesh = plsc.VectorSubcoreMesh(core_axis_name="c", subcore_axis_name="s")


@functools.partial(
    pl.kernel,
    out_type=jax.ShapeDtypeStruct((NC, NPAD, D), jnp.float32),
    mesh=_mesh,
    scratch_types=[
        pltpu.VMEM((2, CH), jnp.int32),
        pltpu.VMEM((CH, D), jnp.float32),
        pltpu.VMEM_SHARED((NPAD, D), jnp.float32),
        pltpu.SemaphoreType.DMA,
    ],
)
def _sc_segsum(hw_hbm, eidx_hbm, zeros_hbm, out_hbm, ei, r0, acc, sg0):
    c = lax.axis_index("c")
    s = lax.axis_index("s")
    wid = s * NC + c

    pltpu.sync_copy(zeros_hbm.at[pl.ds(s * RPT, RPT)], acc.at[pl.ds(s * RPT, RPT)])
    plsc.subcore_barrier()

    def body(j, carry):
        pltpu.sync_copy(eidx_hbm.at[wid, j], ei)
        pltpu.async_copy(hw_hbm.at[ei.at[0]], r0, sg0).wait()
        pltpu.sync_copy(r0, acc.at[ei.at[1]], add=True)
        return carry

    lax.fori_loop(0, NCH, body, 0)
    plsc.subcore_barrier()
    pltpu.sync_copy(acc.at[pl.ds(s * RPT, RPT)], out_hbm.at[c, pl.ds(s * RPT, RPT)])


@functools.partial(
    pl.kernel,
    out_type=[
        jax.ShapeDtypeStruct((B, D), jnp.float32),
        jax.ShapeDtypeStruct((B, D), jnp.float32),
    ],
    mesh=_mesh,
    scratch_types=[
        pltpu.VMEM((BPW,), jnp.int32),
        pltpu.VMEM((BPW,), jnp.int32),
        pltpu.VMEM((BPW, D), jnp.float32),
        pltpu.VMEM((BPW, D), jnp.float32),
        pltpu.SemaphoreType.DMA,
    ],
)
def _sc_qgather(h_hbm, erel_hbm, e1_hbm, rel_hbm, o1_hbm, o2_hbm,
                i1_v, i2_v, r1_v, r2_v, sem):
    c = lax.axis_index("c")
    s = lax.axis_index("s")
    wid = s * NC + c
    pltpu.sync_copy(e1_hbm.at[wid], i1_v)
    pltpu.sync_copy(rel_hbm.at[wid], i2_v)
    pltpu.async_copy(h_hbm.at[i1_v], r1_v, sem).wait()
    pltpu.async_copy(erel_hbm.at[i2_v], r2_v, sem).wait()
    pltpu.sync_copy(r1_v, o1_hbm.at[pl.ds(wid * BPW, BPW)])
    pltpu.sync_copy(r2_v, o2_hbm.at[pl.ds(wid * BPW, BPW)])


def _mm_body(x_ref, w_ref, o_ref):
    o_ref[...] = jnp.dot(x_ref[...], w_ref[...],
                         preferred_element_type=jnp.float32)


def _tc_matmul(x, w):
    n = x.shape[0]
    return pl.pallas_call(
        _mm_body,
        grid=(n // ROWBLK,),
        in_specs=[
            pl.BlockSpec((ROWBLK, D), lambda i: (i, 0)),
            pl.BlockSpec((D, D), lambda i: (0, 0)),
        ],
        out_specs=pl.BlockSpec((ROWBLK, D), lambda i: (i, 0)),
        out_shape=jax.ShapeDtypeStruct((n, D), jnp.float32),
    )(x, w)


def _gru_body(p_ref, h_ref, wih_ref, whh_ref, bih_ref, bhh_ref, wmsg_ref,
              h_out, hw_out):
    a = p_ref[0] + p_ref[1]
    h = h_ref[...]
    gi = jnp.dot(a, wih_ref[...], preferred_element_type=jnp.float32) + bih_ref[...]
    gh = jnp.dot(h, whh_ref[...], preferred_element_type=jnp.float32) + bhh_ref[...]
    r = jax.nn.sigmoid(gi[:, :D] + gh[:, :D])
    z = jax.nn.sigmoid(gi[:, D:2 * D] + gh[:, D:2 * D])
    n = jnp.tanh(gi[:, 2 * D:] + r * gh[:, 2 * D:])
    hn = (1.0 - z) * n + z * h
    h_out[...] = hn
    hw_out[...] = jnp.dot(hn, wmsg_ref[...], preferred_element_type=jnp.float32)


def _tc_gru(parts, h, w_ih, w_hh, b_ih, b_hh, w_msg):
    return pl.pallas_call(
        _gru_body,
        grid=(N_ENT // ROWBLK,),
        in_specs=[
            pl.BlockSpec((NC, ROWBLK, D), lambda i: (0, i, 0)),
            pl.BlockSpec((ROWBLK, D), lambda i: (i, 0)),
            pl.BlockSpec((D, 3 * D), lambda i: (0, 0)),
            pl.BlockSpec((D, 3 * D), lambda i: (0, 0)),
            pl.BlockSpec((1, 3 * D), lambda i: (0, 0)),
            pl.BlockSpec((1, 3 * D), lambda i: (0, 0)),
            pl.BlockSpec((D, D), lambda i: (0, 0)),
        ],
        out_specs=[
            pl.BlockSpec((ROWBLK, D), lambda i: (i, 0)),
            pl.BlockSpec((ROWBLK, D), lambda i: (i, 0)),
        ],
        out_shape=[
            jax.ShapeDtypeStruct((N_ENT, D), jnp.float32),
            jax.ShapeDtypeStruct((N_ENT, D), jnp.float32),
        ],
    )(parts, h, w_ih, w_hh, b_ih, b_hh, w_msg)


def _score_body(q1_ref, q2_ref, e_ref, o_ref):
    q = q1_ref[...] * q2_ref[...]
    o_ref[...] = jax.nn.sigmoid(
        lax.dot_general(q, e_ref[...], (((1,), (1,)), ((), ())),
                        preferred_element_type=jnp.float32))


def _tc_score(q1, q2, emb_pad):
    return pl.pallas_call(
        _score_body,
        grid=(NEPAD // EBLK,),
        in_specs=[
            pl.BlockSpec((B, D), lambda i: (0, 0)),
            pl.BlockSpec((B, D), lambda i: (0, 0)),
            pl.BlockSpec((EBLK, D), lambda i: (i, 0)),
        ],
        out_specs=pl.BlockSpec((B, EBLK), lambda i: (0, i)),
        out_shape=jax.ShapeDtypeStruct((B, NEPAD), jnp.float32),
    )(q1, q2, emb_pad)


def kernel(e1, rel, edge_index, emb_e, emb_rel, W_msg, W_ih, W_hh, b_ih, b_hh):
    src = edge_index[0].astype(jnp.int32)
    dst = edge_index[1].astype(jnp.int32)
    npad = EPAD - E
    srcp = jnp.concatenate([src, jnp.zeros((npad,), jnp.int32)]).reshape(NW, NCH, 1, CH)
    dstp = jnp.concatenate([dst, jnp.full((npad,), N_ENT, jnp.int32)]).reshape(NW, NCH, 1, CH)
    eidx = jnp.concatenate([srcp, dstp], axis=2)
    zeros = jnp.zeros((NPAD, D), jnp.float32)
    bih2 = b_ih.reshape(1, 3 * D)
    bhh2 = b_hh.reshape(1, 3 * D)
    e1i = e1[:, 0].astype(jnp.int32).reshape(NW, BPW)
    reli = rel[:, 0].astype(jnp.int32).reshape(NW, BPW)

    h = emb_e
    hw = _tc_matmul(h, W_msg)
    for _ in range(2):
        parts = _sc_segsum(hw, eidx, zeros)
        h, hw = _tc_gru(parts[:, :N_ENT, :], h, W_ih, W_hh, bih2, bhh2, W_msg)

    e1r, relr = _sc_qgather(h, emb_rel, e1i, reli)
    emb_pad = jnp.concatenate(
        [emb_e, jnp.zeros((NEPAD - N_ENT, D), jnp.float32)], axis=0)
    return _tc_score(e1r, relr, emb_pad)[:, :N_ENT]
